# baseline (speedup 1.0000x reference)
import functools

import jax
import jax.numpy as jnp
from jax import lax
from jax.experimental import pallas as pl
from jax.experimental.pallas import tpu as pltpu
from jax.experimental.pallas import tpu_sc as plsc

N = 10000
E = 320000
D = 128

NC = 2
NS = 16
NW = NC * NS
EPW = E // NW
KC = 80
KN = 100
NP = 10240
RPT = NP // NS

_SC_PARAMS = pltpu.CompilerParams(use_tc_tiling_on_sc=False,
                                  needs_layout_passes=False)
ZCH = 80


def _zero_agg(rows, agg_sh, s):
    lanes = 32 if rows.dtype == jnp.bfloat16 else 16
    zv = jnp.zeros((lanes,), rows.dtype)

    def zrow(r, carry):
        for g in range(D // lanes):
            rows[r, pl.ds(g * lanes, lanes)] = zv
        return carry

    lax.fori_loop(0, ZCH, zrow, 0)
    zsrc = rows.at[pl.ds(0, ZCH)]
    for k in range(RPT // ZCH):
        pltpu.sync_copy(zsrc, agg_sh.at[pl.ds(s * RPT + k * ZCH, ZCH)])


@functools.lru_cache(maxsize=None)
def _make_sc_agg(with_cnt: bool):
    K = KC
    NCHUNK = EPW // K
    NB = 3
    NGRP = (NCHUNK - 5) // NB
    out_type = [jax.ShapeDtypeStruct((NC, NP, D), jnp.bfloat16)]
    scratch = (
        [pltpu.VMEM((K,), jnp.int32) for _ in range(NB)]
        + [pltpu.VMEM((K,), jnp.int32) for _ in range(NB)]
        + [pltpu.VMEM((K, D), jnp.bfloat16) for _ in range(NB)]
        + [pltpu.VMEM_SHARED((NP, D), jnp.bfloat16)]
        + [pltpu.SemaphoreType.DMA] * (4 * NB)
    )
    if with_cnt:
        out_type.append(jax.ShapeDtypeStruct((NW, N), jnp.float32))
        scratch.append(pltpu.VMEM((N,), jnp.float32))

    def body(src_hbm, dst_hbm, y_hbm, agg_out, *rest):
        if with_cnt:
            cnt_out = rest[0]
            rest = rest[1:]
        sb = rest[0:NB]
        db = rest[NB:2 * NB]
        rows = rest[2 * NB:3 * NB]
        agg_sh = rest[3 * NB]
        gsem = rest[3 * NB + 1:3 * NB + 1 + NB]
        ssem = rest[3 * NB + 1 + NB:3 * NB + 1 + 2 * NB]
        iss = rest[3 * NB + 1 + 2 * NB:3 * NB + 1 + 3 * NB]
        isd = rest[3 * NB + 1 + 3 * NB:3 * NB + 1 + 4 * NB]
        if with_cnt:
            cnt_v = rest[3 * NB + 1 + 4 * NB]
        c = lax.axis_index("c")
        s = lax.axis_index("s")
        wid = c * NS + s
        ebase = wid * EPW

        def start_is(i, b):
            pltpu.async_copy(src_hbm.at[pl.ds(ebase + i * K, K)], sb[b], iss[b])

        def wait_is(i, b):
            pltpu.make_async_copy(src_hbm.at[pl.ds(ebase + i * K, K)],
                                  sb[b], iss[b]).wait()

        def start_id(i, b):
            pltpu.async_copy(dst_hbm.at[pl.ds(ebase + i * K, K)], db[b], isd[b])

        def wait_id(i, b):
            pltpu.make_async_copy(dst_hbm.at[pl.ds(ebase + i * K, K)],
                                  db[b], isd[b]).wait()

        def start_g(b):
            pltpu.async_copy(y_hbm.at[sb[b]], rows[b], gsem[b])

        def wait_g(b):
            pltpu.make_async_copy(y_hbm.at[sb[b]], rows[b], gsem[b]).wait()

        def start_scat(b):
            pltpu.async_copy(rows[b], agg_sh.at[db[b]], ssem[b], add=True)

        def wait_scat(b):
            pltpu.make_async_copy(rows[b], agg_sh.at[db[b]], ssem[b]).wait()

        ones16 = jnp.ones((16,), jnp.float32)

        def cnt_upd(b):
            if with_cnt:
                for g in range(K // 16):
                    plsc.addupdate_scatter(cnt_v, [db[b][pl.ds(g * 16, 16)]],
                                           ones16)

        for b in range(NB):
            start_is(b, b)
        start_id(0, 0)
        start_id(1, 1)
        if with_cnt:
            z16 = jnp.zeros((16,), jnp.float32)

            def zcnt(t, carry):
                cnt_v[pl.ds(t * 16, 16)] = z16
                return carry

            lax.fori_loop(0, N // 16, zcnt, 0)
        _zero_agg(rows[0], agg_sh, s)
        wait_is(0, 0)
        start_g(0)
        wait_is(1, 1)
        start_g(1)
        plsc.subcore_barrier()

        def slot(i, b0, first=False):
            b2 = (b0 + 2) % NB
            wait_g(b0)
            wait_id(i, b0)
            cnt_upd(b0)
            start_scat(b0)
            start_is(i + NB, b0)
            if not first:
                wait_scat(b2)
            start_id(i + 2, b2)
            wait_is(i + 2, b2)
            start_g(b2)

        slot(0, 0, first=True)
        slot(1, 1)

        def grp(g, carry):
            base = NB * g + 2
            for q in range(NB):
                slot(base + q, (2 + q) % NB)
            return carry

        lax.fori_loop(0, NGRP, grp, 0)

        i = NCHUNK - 3
        wait_g(2)
        wait_id(i, 2)
        cnt_upd(2)
        start_scat(2)
        wait_scat(1)
        start_id(i + 2, 1)
        wait_is(i + 2, 1)
        start_g(1)
        wait_g(0)
        wait_id(i + 1, 0)
        cnt_upd(0)
        start_scat(0)
        wait_scat(2)
        wait_g(1)
        wait_id(i + 2, 1)
        cnt_upd(1)
        start_scat(1)
        wait_scat(0)
        wait_scat(1)
        plsc.subcore_barrier()

        sl = pl.ds(s * RPT, RPT)
        pltpu.sync_copy(agg_sh.at[sl], agg_out.at[c, sl])
        if with_cnt:
            pltpu.sync_copy(cnt_v, cnt_out.at[wid])

    ot = out_type if with_cnt else out_type[0]
    mesh = plsc.VectorSubcoreMesh(core_axis_name="c", subcore_axis_name="s")
    return pl.kernel(body, mesh=mesh, out_type=ot,
                     scratch_types=scratch, compiler_params=_SC_PARAMS)


@functools.lru_cache(maxsize=None)
def _make_sc_agg4():
    K = KC
    NCHUNK = EPW // K
    NB = 4
    out_type = jax.ShapeDtypeStruct((NC, NP, D), jnp.float32)
    scratch = (
        [pltpu.VMEM((K,), jnp.int32) for _ in range(NB)]
        + [pltpu.VMEM((K,), jnp.int32) for _ in range(NB)]
        + [pltpu.VMEM((K, D), jnp.float32) for _ in range(NB)]
        + [pltpu.VMEM_SHARED((NP, D), jnp.float32)]
        + [pltpu.SemaphoreType.DMA] * (4 * NB)
    )

    def body(src_hbm, dst_hbm, y_hbm, agg_out, *rest):
        sb = rest[0:NB]
        db = rest[NB:2 * NB]
        rows = rest[2 * NB:3 * NB]
        agg_sh = rest[3 * NB]
        sems = rest[3 * NB + 1:]
        gsem = sems[0:NB]
        ssem = sems[NB:2 * NB]
        iss = sems[2 * NB:3 * NB]
        isd = sems[3 * NB:4 * NB]
        c = lax.axis_index("c")
        s = lax.axis_index("s")
        wid = c * NS + s
        ebase = wid * EPW

        def start_is(i, b):
            pltpu.async_copy(src_hbm.at[pl.ds(ebase + i * K, K)], sb[b], iss[b])

        def wait_is(i, b):
            pltpu.make_async_copy(src_hbm.at[pl.ds(ebase + i * K, K)],
                                  sb[b], iss[b]).wait()

        def start_id(i, b):
            pltpu.async_copy(dst_hbm.at[pl.ds(ebase + i * K, K)], db[b], isd[b])

        def wait_id(i, b):
            pltpu.make_async_copy(dst_hbm.at[pl.ds(ebase + i * K, K)],
                                  db[b], isd[b]).wait()

        def start_g(b):
            pltpu.async_copy(y_hbm.at[sb[b]], rows[b], gsem[b])

        def wait_g(b):
            pltpu.make_async_copy(y_hbm.at[sb[b]], rows[b], gsem[b]).wait()

        def start_scat(b):
            pltpu.async_copy(rows[b], agg_sh.at[db[b]], ssem[b], add=True)

        def wait_scat(b):
            pltpu.make_async_copy(rows[b], agg_sh.at[db[b]], ssem[b]).wait()

        for b in range(NB):
            start_is(b, b)
        start_id(0, 0)
        start_id(1, 1)
        _zero_agg(rows[0], agg_sh, s)
        wait_is(0, 0)
        start_g(0)
        wait_is(1, 1)
        start_g(1)
        plsc.subcore_barrier()

        def slot(i, b0, w_scat=True, do_is=True, do_id=True, do_g=True):
            bg = (b0 + 2) % NB
            wait_g(b0)
            wait_id(i, b0)
            start_scat(b0)
            if do_is:
                start_is(i + NB, b0)
            if w_scat:
                wait_scat(bg)
            if do_id:
                start_id(i + 2, bg)
            if do_g:
                wait_is(i + 2, bg)
                start_g(bg)

        slot(0, 0, w_scat=False)
        slot(1, 1, w_scat=False)
        slot(2, 2)
        slot(3, 3)

        def grp(g, carry):
            base = NB * g + 4
            for q in range(NB):
                slot(base + q, q)
            return carry

        lax.fori_loop(0, (NCHUNK - 9) // NB, grp, 0)

        slot(120, 0)
        slot(121, 1, do_is=False)
        slot(122, 2, do_is=False)
        slot(123, 3, do_is=False, do_id=False, do_g=False)
        slot(124, 0, do_is=False, do_id=False, do_g=False)
        wait_scat(3)
        wait_scat(0)
        plsc.subcore_barrier()

        sl = pl.ds(s * RPT, RPT)
        pltpu.sync_copy(agg_sh.at[sl], agg_out.at[c, sl])

    mesh = plsc.VectorSubcoreMesh(core_axis_name="c", subcore_axis_name="s")
    return pl.kernel(body, mesh=mesh, out_type=out_type,
                     scratch_types=scratch, compiler_params=_SC_PARAMS)


BR = 2000


def _ln_relu(x, g, b):
    m = jnp.mean(x, axis=-1, keepdims=True)
    v = jnp.mean((x - m) ** 2, axis=-1, keepdims=True)
    h = (x - m) * lax.rsqrt(v + 1e-5) * g + b
    return jnp.maximum(h, 0.0)


def _tc_pre_body(x_ref, g_ref, b_ref, wlt_ref, y_ref):
    h = _ln_relu(x_ref[...], g_ref[...], b_ref[...])
    y_ref[...] = jnp.dot(h, wlt_ref[...],
                         preferred_element_type=jnp.float32).astype(y_ref.dtype)


def _tc_mm_body(h_ref, w_ref, o_ref):
    o_ref[...] = jnp.dot(h_ref[...], w_ref[...],
                         preferred_element_type=jnp.float32)


def _mean_from_partials(agg, cnt):
    a = agg[0].astype(jnp.float32) + agg[1].astype(jnp.float32)
    c = jnp.sum(cnt, axis=-1, keepdims=True)
    return a / jnp.maximum(c, 1.0)


def _tc_mid_body(agg_ref, cnt_ref, s0_ref, b0l_ref, g_ref, b_ref,
                 wlt_ref, y_ref, h_ref):
    t = _mean_from_partials(agg_ref[...], cnt_ref[...]) + b0l_ref[...] + s0_ref[...]
    h = _ln_relu(t, g_ref[...], b_ref[...])
    y_ref[...] = jnp.dot(h, wlt_ref[...],
                         preferred_element_type=jnp.float32).astype(y_ref.dtype)
    h_ref[...] = h


def _tc_post_body(x_ref, agg_ref, cnt_ref, s1_ref, b1l_ref, out_ref):
    t = _mean_from_partials(agg_ref[...], cnt_ref[...]) + b1l_ref[...] + s1_ref[...]
    out_ref[...] = x_ref[...] + t


_row_spec = pl.BlockSpec((BR, D), lambda i: (i, 0))
_vec_spec = pl.BlockSpec((1, D), lambda i: (0, 0))
_w_spec = pl.BlockSpec((D, D), lambda i: (0, 0))
_agg_spec = pl.BlockSpec((NC, BR, D), lambda i: (0, i, 0))
_cnt_spec = pl.BlockSpec((BR, NW), lambda i: (i, 0))
_GRID = (N // BR,)

_tc_pre = pl.pallas_call(
    _tc_pre_body,
    grid=_GRID,
    in_specs=[_row_spec, _vec_spec, _vec_spec, _w_spec],
    out_specs=_row_spec,
    out_shape=jax.ShapeDtypeStruct((N, D), jnp.bfloat16),
)

_tc_preh = pl.pallas_call(
    _tc_pre_body,
    grid=_GRID,
    in_specs=[_row_spec, _vec_spec, _vec_spec, _w_spec],
    out_specs=_row_spec,
    out_shape=jax.ShapeDtypeStruct((N, D), jnp.float32),
)

_tc_mm = pl.pallas_call(
    _tc_mm_body,
    grid=_GRID,
    in_specs=[_row_spec, _w_spec],
    out_specs=_row_spec,
    out_shape=jax.ShapeDtypeStruct((N, D), jnp.float32),
)

_tc_mid = pl.pallas_call(
    _tc_mid_body,
    grid=_GRID,
    in_specs=[_agg_spec, _cnt_spec, _row_spec, _vec_spec, _vec_spec,
              _vec_spec, _w_spec],
    out_specs=[_row_spec, _row_spec],
    out_shape=[jax.ShapeDtypeStruct((N, D), jnp.bfloat16),
               jax.ShapeDtypeStruct((N, D), jnp.float32)],
)

_tc_post = pl.pallas_call(
    _tc_post_body,
    grid=_GRID,
    in_specs=[_row_spec, _agg_spec, _cnt_spec, _row_spec, _vec_spec],
    out_specs=_row_spec,
    out_shape=jax.ShapeDtypeStruct((N, D), jnp.float32),
)


def kernel(x, edge_index, ln0_g, ln0_b, w0l, b0l, w0r, ln1_g, ln1_b,
           w1l, b1l, w1r):
    src = edge_index[0]
    dst = edge_index[1]
    g0 = ln0_g.reshape(1, D)
    b0 = ln0_b.reshape(1, D)
    g1 = ln1_g.reshape(1, D)
    b1 = ln1_b.reshape(1, D)
    b0l2 = b0l.reshape(1, D)
    b1l2 = b1l.reshape(1, D)
    y0 = _tc_pre(x, g0, b0, w0l.T)
    agg0, cnt_p = _make_sc_agg(True)(src, dst, y0)
    s0 = _tc_preh(x, g0, b0, w0r.T)
    cnt = cnt_p.T
    y1, h1 = _tc_mid(agg0, cnt, s0, b0l2, g1, b1, w1l.T)
    agg1 = _make_sc_agg(False)(src, dst, y1)
    s1 = _tc_mm(h1, w1r.T)
    return _tc_post(x, agg1, cnt, s1, b1l2)


# speedup vs baseline: 1.0007x; 1.0007x over previous
import functools

import jax
import jax.numpy as jnp
from jax import lax
from jax.experimental import pallas as pl
from jax.experimental.pallas import tpu as pltpu
from jax.experimental.pallas import tpu_sc as plsc

N = 10000
E = 320000
D = 128

NC = 2
NS = 16
NW = NC * NS
EPW = E // NW
KC = 80
KN = 100
NP = 10240
RPT = NP // NS

_SC_PARAMS = pltpu.CompilerParams(use_tc_tiling_on_sc=False,
                                  needs_layout_passes=False)
ZCH = 80


def _zero_agg(rows, agg_sh, s):
    z16 = jnp.zeros((16,), jnp.float32)

    def zrow(r, carry):
        for g in range(D // 16):
            rows[r, pl.ds(g * 16, 16)] = z16
        return carry

    lax.fori_loop(0, ZCH, zrow, 0)
    zsrc = rows.at[pl.ds(0, ZCH)]
    for k in range(RPT // ZCH):
        pltpu.sync_copy(zsrc, agg_sh.at[pl.ds(s * RPT + k * ZCH, ZCH)])


@functools.lru_cache(maxsize=None)
def _make_sc_agg(with_cnt: bool):
    K = KC
    NCHUNK = EPW // K
    NB = 3
    NGRP = (NCHUNK - 5) // NB
    out_type = [jax.ShapeDtypeStruct((NC, NP, D), jnp.float32)]
    scratch = (
        [pltpu.VMEM((K,), jnp.int32) for _ in range(NB)]
        + [pltpu.VMEM((K,), jnp.int32) for _ in range(NB)]
        + [pltpu.VMEM((K, D), jnp.float32) for _ in range(NB)]
        + [pltpu.VMEM_SHARED((NP, D), jnp.float32)]
        + [pltpu.SemaphoreType.DMA] * (4 * NB)
    )
    if with_cnt:
        out_type.append(jax.ShapeDtypeStruct((NW, N), jnp.float32))
        scratch.append(pltpu.VMEM((N,), jnp.float32))

    def body(src_hbm, dst_hbm, y_hbm, agg_out, *rest):
        if with_cnt:
            cnt_out = rest[0]
            rest = rest[1:]
        sb = rest[0:NB]
        db = rest[NB:2 * NB]
        rows = rest[2 * NB:3 * NB]
        agg_sh = rest[3 * NB]
        gsem = rest[3 * NB + 1:3 * NB + 1 + NB]
        ssem = rest[3 * NB + 1 + NB:3 * NB + 1 + 2 * NB]
        iss = rest[3 * NB + 1 + 2 * NB:3 * NB + 1 + 3 * NB]
        isd = rest[3 * NB + 1 + 3 * NB:3 * NB + 1 + 4 * NB]
        if with_cnt:
            cnt_v = rest[3 * NB + 1 + 4 * NB]
        c = lax.axis_index("c")
        s = lax.axis_index("s")
        wid = c * NS + s
        ebase = wid * EPW

        def start_is(i, b):
            pltpu.async_copy(src_hbm.at[pl.ds(ebase + i * K, K)], sb[b], iss[b])

        def wait_is(i, b):
            pltpu.make_async_copy(src_hbm.at[pl.ds(ebase + i * K, K)],
                                  sb[b], iss[b]).wait()

        def start_id(i, b):
            pltpu.async_copy(dst_hbm.at[pl.ds(ebase + i * K, K)], db[b], isd[b])

        def wait_id(i, b):
            pltpu.make_async_copy(dst_hbm.at[pl.ds(ebase + i * K, K)],
                                  db[b], isd[b]).wait()

        def start_g(b):
            pltpu.async_copy(y_hbm.at[sb[b]], rows[b], gsem[b])

        def wait_g(b):
            pltpu.make_async_copy(y_hbm.at[sb[b]], rows[b], gsem[b]).wait()

        def start_scat(b):
            pltpu.async_copy(rows[b], agg_sh.at[db[b]], ssem[b], add=True)

        def wait_scat(b):
            pltpu.make_async_copy(rows[b], agg_sh.at[db[b]], ssem[b]).wait()

        ones16 = jnp.ones((16,), jnp.float32)

        def cnt_upd(b):
            if with_cnt:
                for g in range(K // 16):
                    plsc.addupdate_scatter(cnt_v, [db[b][pl.ds(g * 16, 16)]],
                                           ones16)

        for b in range(NB):
            start_is(b, b)
        start_id(0, 0)
        start_id(1, 1)
        if with_cnt:
            z16 = jnp.zeros((16,), jnp.float32)

            def zcnt(t, carry):
                cnt_v[pl.ds(t * 16, 16)] = z16
                return carry

            lax.fori_loop(0, N // 16, zcnt, 0)
        _zero_agg(rows[0], agg_sh, s)
        wait_is(0, 0)
        start_g(0)
        wait_is(1, 1)
        start_g(1)
        plsc.subcore_barrier()

        def slot(i, b0, first=False):
            b2 = (b0 + 2) % NB
            wait_g(b0)
            wait_id(i, b0)
            cnt_upd(b0)
            start_scat(b0)
            start_is(i + NB, b0)
            if not first:
                wait_scat(b2)
            start_id(i + 2, b2)
            wait_is(i + 2, b2)
            start_g(b2)

        slot(0, 0, first=True)
        slot(1, 1)

        def grp(g, carry):
            base = NB * g + 2
            for q in range(NB):
                slot(base + q, (2 + q) % NB)
            return carry

        lax.fori_loop(0, NGRP, grp, 0)

        i = NCHUNK - 3
        wait_g(2)
        wait_id(i, 2)
        cnt_upd(2)
        start_scat(2)
        wait_scat(1)
        start_id(i + 2, 1)
        wait_is(i + 2, 1)
        start_g(1)
        wait_g(0)
        wait_id(i + 1, 0)
        cnt_upd(0)
        start_scat(0)
        wait_scat(2)
        wait_g(1)
        wait_id(i + 2, 1)
        cnt_upd(1)
        start_scat(1)
        wait_scat(0)
        wait_scat(1)
        plsc.subcore_barrier()

        sl = pl.ds(s * RPT, RPT)
        pltpu.sync_copy(agg_sh.at[sl], agg_out.at[c, sl])
        if with_cnt:
            pltpu.sync_copy(cnt_v, cnt_out.at[wid])

    ot = out_type if with_cnt else out_type[0]
    mesh = plsc.VectorSubcoreMesh(core_axis_name="c", subcore_axis_name="s")
    return pl.kernel(body, mesh=mesh, out_type=ot,
                     scratch_types=scratch, compiler_params=_SC_PARAMS)


@functools.lru_cache(maxsize=None)
def _make_sc_agg4():
    K = KC
    NCHUNK = EPW // K
    NB = 4
    out_type = jax.ShapeDtypeStruct((NC, NP, D), jnp.float32)
    scratch = (
        [pltpu.VMEM((K,), jnp.int32) for _ in range(NB)]
        + [pltpu.VMEM((K,), jnp.int32) for _ in range(NB)]
        + [pltpu.VMEM((K, D), jnp.float32) for _ in range(NB)]
        + [pltpu.VMEM_SHARED((NP, D), jnp.float32)]
        + [pltpu.SemaphoreType.DMA] * (4 * NB)
    )

    def body(src_hbm, dst_hbm, y_hbm, agg_out, *rest):
        sb = rest[0:NB]
        db = rest[NB:2 * NB]
        rows = rest[2 * NB:3 * NB]
        agg_sh = rest[3 * NB]
        sems = rest[3 * NB + 1:]
        gsem = sems[0:NB]
        ssem = sems[NB:2 * NB]
        iss = sems[2 * NB:3 * NB]
        isd = sems[3 * NB:4 * NB]
        c = lax.axis_index("c")
        s = lax.axis_index("s")
        wid = c * NS + s
        ebase = wid * EPW

        def start_is(i, b):
            pltpu.async_copy(src_hbm.at[pl.ds(ebase + i * K, K)], sb[b], iss[b])

        def wait_is(i, b):
            pltpu.make_async_copy(src_hbm.at[pl.ds(ebase + i * K, K)],
                                  sb[b], iss[b]).wait()

        def start_id(i, b):
            pltpu.async_copy(dst_hbm.at[pl.ds(ebase + i * K, K)], db[b], isd[b])

        def wait_id(i, b):
            pltpu.make_async_copy(dst_hbm.at[pl.ds(ebase + i * K, K)],
                                  db[b], isd[b]).wait()

        def start_g(b):
            pltpu.async_copy(y_hbm.at[sb[b]], rows[b], gsem[b])

        def wait_g(b):
            pltpu.make_async_copy(y_hbm.at[sb[b]], rows[b], gsem[b]).wait()

        def start_scat(b):
            pltpu.async_copy(rows[b], agg_sh.at[db[b]], ssem[b], add=True)

        def wait_scat(b):
            pltpu.make_async_copy(rows[b], agg_sh.at[db[b]], ssem[b]).wait()

        for b in range(NB):
            start_is(b, b)
        start_id(0, 0)
        start_id(1, 1)
        _zero_agg(rows[0], agg_sh, s)
        wait_is(0, 0)
        start_g(0)
        wait_is(1, 1)
        start_g(1)
        plsc.subcore_barrier()

        def slot(i, b0, w_scat=True, do_is=True, do_id=True, do_g=True):
            bg = (b0 + 2) % NB
            wait_g(b0)
            wait_id(i, b0)
            start_scat(b0)
            if do_is:
                start_is(i + NB, b0)
            if w_scat:
                wait_scat(bg)
            if do_id:
                start_id(i + 2, bg)
            if do_g:
                wait_is(i + 2, bg)
                start_g(bg)

        slot(0, 0, w_scat=False)
        slot(1, 1, w_scat=False)
        slot(2, 2)
        slot(3, 3)

        def grp(g, carry):
            base = NB * g + 4
            for q in range(NB):
                slot(base + q, q)
            return carry

        lax.fori_loop(0, (NCHUNK - 9) // NB, grp, 0)

        slot(120, 0)
        slot(121, 1, do_is=False)
        slot(122, 2, do_is=False)
        slot(123, 3, do_is=False, do_id=False, do_g=False)
        slot(124, 0, do_is=False, do_id=False, do_g=False)
        wait_scat(3)
        wait_scat(0)
        plsc.subcore_barrier()

        sl = pl.ds(s * RPT, RPT)
        pltpu.sync_copy(agg_sh.at[sl], agg_out.at[c, sl])

    mesh = plsc.VectorSubcoreMesh(core_axis_name="c", subcore_axis_name="s")
    return pl.kernel(body, mesh=mesh, out_type=out_type,
                     scratch_types=scratch, compiler_params=_SC_PARAMS)


BR = 2000


def _ln_relu(x, g, b):
    m = jnp.mean(x, axis=-1, keepdims=True)
    v = jnp.mean((x - m) ** 2, axis=-1, keepdims=True)
    h = (x - m) * lax.rsqrt(v + 1e-5) * g + b
    return jnp.maximum(h, 0.0)


def _tc_pre_body(x_ref, g_ref, b_ref, wlt_ref, y_ref):
    h = _ln_relu(x_ref[...], g_ref[...], b_ref[...])
    y_ref[...] = jnp.dot(h, wlt_ref[...], preferred_element_type=jnp.float32)


def _tc_mm_body(h_ref, w_ref, o_ref):
    o_ref[...] = jnp.dot(h_ref[...], w_ref[...],
                         preferred_element_type=jnp.float32)


def _mean_from_partials(agg, cnt):
    a = agg[0] + agg[1]
    c = jnp.sum(cnt, axis=-1, keepdims=True)
    return a / jnp.maximum(c, 1.0)


def _tc_mid_body(agg_ref, cnt_ref, s0_ref, b0l_ref, g_ref, b_ref,
                 wlt_ref, y_ref, h_ref):
    t = _mean_from_partials(agg_ref[...], cnt_ref[...]) + b0l_ref[...] + s0_ref[...]
    h = _ln_relu(t, g_ref[...], b_ref[...])
    y_ref[...] = jnp.dot(h, wlt_ref[...], preferred_element_type=jnp.float32)
    h_ref[...] = h


def _tc_post_body(x_ref, agg_ref, cnt_ref, s1_ref, b1l_ref, out_ref):
    t = _mean_from_partials(agg_ref[...], cnt_ref[...]) + b1l_ref[...] + s1_ref[...]
    out_ref[...] = x_ref[...] + t


_row_spec = pl.BlockSpec((BR, D), lambda i: (i, 0))
_vec_spec = pl.BlockSpec((1, D), lambda i: (0, 0))
_w_spec = pl.BlockSpec((D, D), lambda i: (0, 0))
_agg_spec = pl.BlockSpec((NC, BR, D), lambda i: (0, i, 0))
_cnt_spec = pl.BlockSpec((BR, NW), lambda i: (i, 0))
_GRID = (N // BR,)

_tc_pre = pl.pallas_call(
    _tc_pre_body,
    grid=_GRID,
    in_specs=[_row_spec, _vec_spec, _vec_spec, _w_spec],
    out_specs=_row_spec,
    out_shape=jax.ShapeDtypeStruct((N, D), jnp.float32),
)

_tc_preh = pl.pallas_call(
    _tc_pre_body,
    grid=_GRID,
    in_specs=[_row_spec, _vec_spec, _vec_spec, _w_spec],
    out_specs=_row_spec,
    out_shape=jax.ShapeDtypeStruct((N, D), jnp.float32),
)

_tc_mm = pl.pallas_call(
    _tc_mm_body,
    grid=_GRID,
    in_specs=[_row_spec, _w_spec],
    out_specs=_row_spec,
    out_shape=jax.ShapeDtypeStruct((N, D), jnp.float32),
)

_tc_mid = pl.pallas_call(
    _tc_mid_body,
    grid=_GRID,
    in_specs=[_agg_spec, _cnt_spec, _row_spec, _vec_spec, _vec_spec,
              _vec_spec, _w_spec],
    out_specs=[_row_spec, _row_spec],
    out_shape=[jax.ShapeDtypeStruct((N, D), jnp.float32)] * 2,
)

_tc_post = pl.pallas_call(
    _tc_post_body,
    grid=_GRID,
    in_specs=[_row_spec, _agg_spec, _cnt_spec, _row_spec, _vec_spec],
    out_specs=_row_spec,
    out_shape=jax.ShapeDtypeStruct((N, D), jnp.float32),
)


def kernel(x, edge_index, ln0_g, ln0_b, w0l, b0l, w0r, ln1_g, ln1_b,
           w1l, b1l, w1r):
    src = edge_index[0]
    dst = edge_index[1]
    g0 = ln0_g.reshape(1, D)
    b0 = ln0_b.reshape(1, D)
    g1 = ln1_g.reshape(1, D)
    b1 = ln1_b.reshape(1, D)
    b0l2 = b0l.reshape(1, D)
    b1l2 = b1l.reshape(1, D)
    y0 = _tc_pre(x, g0, b0, w0l.T)
    agg0, cnt_p = _make_sc_agg(True)(src, dst, y0)
    s0 = _tc_preh(x, g0, b0, w0r.T)
    cnt = cnt_p.T
    y1, h1 = _tc_mid(agg0, cnt, s0, b0l2, g1, b1, w1l.T)
    agg1 = _make_sc_agg(False)(src, dst, y1)
    s1 = _tc_mm(h1, w1r.T)
    return _tc_post(x, agg1, cnt, s1, b1l2)


# speedup vs baseline: 1.0015x; 1.0008x over previous
import functools

import jax
import jax.numpy as jnp
from jax import lax
from jax.experimental import pallas as pl
from jax.experimental.pallas import tpu as pltpu
from jax.experimental.pallas import tpu_sc as plsc

N = 10000
E = 320000
D = 128

NC = 2
NS = 16
NW = NC * NS
EPW = E // NW
KC = 80
KN = 100
NP = 10240
RPT = NP // NS

_SC_PARAMS = pltpu.CompilerParams(use_tc_tiling_on_sc=False,
                                  needs_layout_passes=False)
ZCH = 80


def _zero_agg(rows, agg_sh, s):
    z16 = jnp.zeros((16,), jnp.float32)

    def zrow(r, carry):
        for g in range(D // 16):
            rows[r, pl.ds(g * 16, 16)] = z16
        return carry

    lax.fori_loop(0, ZCH, zrow, 0)
    zsrc = rows.at[pl.ds(0, ZCH)]
    for k in range(RPT // ZCH):
        pltpu.sync_copy(zsrc, agg_sh.at[pl.ds(s * RPT + k * ZCH, ZCH)])


@functools.lru_cache(maxsize=None)
def _make_sc_agg(with_cnt: bool):
    K = KC
    NCHUNK = EPW // K
    NB = 3
    NGRP = (NCHUNK - 5) // NB
    out_type = [jax.ShapeDtypeStruct((NC, NP, D), jnp.float32)]
    scratch = (
        [pltpu.VMEM((K,), jnp.int32) for _ in range(NB)]
        + [pltpu.VMEM((K,), jnp.int32) for _ in range(NB)]
        + [pltpu.VMEM((K, D), jnp.float32) for _ in range(NB)]
        + [pltpu.VMEM_SHARED((NP, D), jnp.float32)]
        + [pltpu.SemaphoreType.DMA] * (4 * NB)
    )
    if with_cnt:
        out_type.append(jax.ShapeDtypeStruct((NW, N), jnp.float32))
        scratch.append(pltpu.VMEM((N,), jnp.float32))

    def body(src_hbm, dst_hbm, y_hbm, agg_out, *rest):
        if with_cnt:
            cnt_out = rest[0]
            rest = rest[1:]
        sb = rest[0:NB]
        db = rest[NB:2 * NB]
        rows = rest[2 * NB:3 * NB]
        agg_sh = rest[3 * NB]
        gsem = rest[3 * NB + 1:3 * NB + 1 + NB]
        ssem = rest[3 * NB + 1 + NB:3 * NB + 1 + 2 * NB]
        iss = rest[3 * NB + 1 + 2 * NB:3 * NB + 1 + 3 * NB]
        isd = rest[3 * NB + 1 + 3 * NB:3 * NB + 1 + 4 * NB]
        if with_cnt:
            cnt_v = rest[3 * NB + 1 + 4 * NB]
        c = lax.axis_index("c")
        s = lax.axis_index("s")
        wid = c * NS + s
        ebase = wid * EPW

        def start_is(i, b):
            pltpu.async_copy(src_hbm.at[pl.ds(ebase + i * K, K)], sb[b], iss[b])

        def wait_is(i, b):
            pltpu.make_async_copy(src_hbm.at[pl.ds(ebase + i * K, K)],
                                  sb[b], iss[b]).wait()

        def start_id(i, b):
            pltpu.async_copy(dst_hbm.at[pl.ds(ebase + i * K, K)], db[b], isd[b])

        def wait_id(i, b):
            pltpu.make_async_copy(dst_hbm.at[pl.ds(ebase + i * K, K)],
                                  db[b], isd[b]).wait()

        def start_g(b):
            pltpu.async_copy(y_hbm.at[sb[b]], rows[b], gsem[b])

        def wait_g(b):
            pltpu.make_async_copy(y_hbm.at[sb[b]], rows[b], gsem[b]).wait()

        def start_scat(b):
            pltpu.async_copy(rows[b], agg_sh.at[db[b]], ssem[b], add=True)

        def wait_scat(b):
            pltpu.make_async_copy(rows[b], agg_sh.at[db[b]], ssem[b]).wait()

        ones16 = jnp.ones((16,), jnp.float32)

        def cnt_upd(b):
            if with_cnt:
                for g in range(K // 16):
                    plsc.addupdate_scatter(cnt_v, [db[b][pl.ds(g * 16, 16)]],
                                           ones16)

        for b in range(NB):
            start_is(b, b)
        start_id(0, 0)
        start_id(1, 1)
        if with_cnt:
            z16 = jnp.zeros((16,), jnp.float32)

            def zcnt(t, carry):
                cnt_v[pl.ds(t * 16, 16)] = z16
                return carry

            lax.fori_loop(0, N // 16, zcnt, 0)
        _zero_agg(rows[0], agg_sh, s)
        wait_is(0, 0)
        start_g(0)
        wait_is(1, 1)
        start_g(1)
        plsc.subcore_barrier()

        def slot(i, b0, first=False):
            b2 = (b0 + 2) % NB
            wait_g(b0)
            wait_id(i, b0)
            cnt_upd(b0)
            start_scat(b0)
            start_is(i + NB, b0)
            if not first:
                wait_scat(b2)
            start_id(i + 2, b2)
            wait_is(i + 2, b2)
            start_g(b2)

        slot(0, 0, first=True)
        slot(1, 1)

        def grp(g, carry):
            base = NB * g + 2
            for q in range(NB):
                slot(base + q, (2 + q) % NB)
            return carry

        lax.fori_loop(0, NGRP, grp, 0)

        i = NCHUNK - 3
        wait_g(2)
        wait_id(i, 2)
        cnt_upd(2)
        start_scat(2)
        wait_scat(1)
        start_id(i + 2, 1)
        wait_is(i + 2, 1)
        start_g(1)
        wait_g(0)
        wait_id(i + 1, 0)
        cnt_upd(0)
        start_scat(0)
        wait_scat(2)
        wait_g(1)
        wait_id(i + 2, 1)
        cnt_upd(1)
        start_scat(1)
        wait_scat(0)
        wait_scat(1)
        plsc.subcore_barrier()

        sl = pl.ds(s * RPT, RPT)
        pltpu.sync_copy(agg_sh.at[sl], agg_out.at[c, sl])
        if with_cnt:
            pltpu.sync_copy(cnt_v, cnt_out.at[wid])

    ot = out_type if with_cnt else out_type[0]
    mesh = plsc.VectorSubcoreMesh(core_axis_name="c", subcore_axis_name="s")
    return pl.kernel(body, mesh=mesh, out_type=ot,
                     scratch_types=scratch, compiler_params=_SC_PARAMS)


BR = 2000


def _ln_relu(x, g, b):
    m = jnp.mean(x, axis=-1, keepdims=True)
    v = jnp.mean((x - m) ** 2, axis=-1, keepdims=True)
    h = (x - m) * lax.rsqrt(v + 1e-5) * g + b
    return jnp.maximum(h, 0.0)


def _tc_pre_body(x_ref, g_ref, b_ref, wlt_ref, y_ref):
    h = _ln_relu(x_ref[...], g_ref[...], b_ref[...])
    y_ref[...] = jnp.dot(h, wlt_ref[...], preferred_element_type=jnp.float32)


def _tc_mm_body(h_ref, w_ref, o_ref):
    o_ref[...] = jnp.dot(h_ref[...], w_ref[...],
                         preferred_element_type=jnp.float32)


def _mean_from_partials(agg, cnt):
    a = agg[0] + agg[1]
    c = jnp.sum(cnt, axis=-1, keepdims=True)
    return a / jnp.maximum(c, 1.0)


def _tc_mid_body(agg_ref, cnt_ref, s0_ref, b0l_ref, g_ref, b_ref,
                 wlt_ref, y_ref, h_ref):
    t = _mean_from_partials(agg_ref[...], cnt_ref[...]) + b0l_ref[...] + s0_ref[...]
    h = _ln_relu(t, g_ref[...], b_ref[...])
    y_ref[...] = jnp.dot(h, wlt_ref[...], preferred_element_type=jnp.float32)
    h_ref[...] = h


def _tc_post_body(x_ref, agg_ref, cnt_ref, s1_ref, b1l_ref, out_ref):
    t = _mean_from_partials(agg_ref[...], cnt_ref[...]) + b1l_ref[...] + s1_ref[...]
    out_ref[...] = x_ref[...] + t


_row_spec = pl.BlockSpec((BR, D), lambda i: (i, 0))
_vec_spec = pl.BlockSpec((1, D), lambda i: (0, 0))
_w_spec = pl.BlockSpec((D, D), lambda i: (0, 0))
_agg_spec = pl.BlockSpec((NC, BR, D), lambda i: (0, i, 0))
_cnt_spec = pl.BlockSpec((BR, NW), lambda i: (i, 0))
_GRID = (N // BR,)

_tc_pre = pl.pallas_call(
    _tc_pre_body,
    grid=_GRID,
    in_specs=[_row_spec, _vec_spec, _vec_spec, _w_spec],
    out_specs=_row_spec,
    out_shape=jax.ShapeDtypeStruct((N, D), jnp.float32),
)

_tc_preh = pl.pallas_call(
    _tc_pre_body,
    grid=_GRID,
    in_specs=[_row_spec, _vec_spec, _vec_spec, _w_spec],
    out_specs=_row_spec,
    out_shape=jax.ShapeDtypeStruct((N, D), jnp.float32),
)

_tc_mm = pl.pallas_call(
    _tc_mm_body,
    grid=_GRID,
    in_specs=[_row_spec, _w_spec],
    out_specs=_row_spec,
    out_shape=jax.ShapeDtypeStruct((N, D), jnp.float32),
)

_tc_mid = pl.pallas_call(
    _tc_mid_body,
    grid=_GRID,
    in_specs=[_agg_spec, _cnt_spec, _row_spec, _vec_spec, _vec_spec,
              _vec_spec, _w_spec],
    out_specs=[_row_spec, _row_spec],
    out_shape=[jax.ShapeDtypeStruct((N, D), jnp.float32)] * 2,
)

_tc_post = pl.pallas_call(
    _tc_post_body,
    grid=_GRID,
    in_specs=[_row_spec, _agg_spec, _cnt_spec, _row_spec, _vec_spec],
    out_specs=_row_spec,
    out_shape=jax.ShapeDtypeStruct((N, D), jnp.float32),
)


def kernel(x, edge_index, ln0_g, ln0_b, w0l, b0l, w0r, ln1_g, ln1_b,
           w1l, b1l, w1r):
    src = edge_index[0]
    dst = edge_index[1]
    g0 = ln0_g.reshape(1, D)
    b0 = ln0_b.reshape(1, D)
    g1 = ln1_g.reshape(1, D)
    b1 = ln1_b.reshape(1, D)
    b0l2 = b0l.reshape(1, D)
    b1l2 = b1l.reshape(1, D)
    y0 = _tc_pre(x, g0, b0, w0l.T)
    agg0, cnt_p = _make_sc_agg(True)(src, dst, y0)
    s0 = _tc_preh(x, g0, b0, w0r.T)
    cnt = cnt_p.T
    y1, h1 = _tc_mid(agg0, cnt, s0, b0l2, g1, b1, w1l.T)
    agg1 = _make_sc_agg(False)(src, dst, y1)
    s1 = _tc_mm(h1, w1r.T)
    return _tc_post(x, agg1, cnt, s1, b1l2)
